# baseline (speedup 1.0000x reference)
import functools

import jax
import jax.numpy as jnp
from jax import lax
from jax.experimental import pallas as pl
from jax.experimental.pallas import tpu as pltpu
from jax.experimental.pallas import tpu_sc as plsc

_CHUNK_ROWS = 32
_N_BUF = 4


def kernel(inputs, embeddings):
    batch = inputs.shape[0]
    seq, emb = embeddings.shape
    info = plsc.get_sparse_core_info()
    num_workers = info.num_cores * info.num_subcores
    rows_per_worker = seq // num_workers
    n_chunks = rows_per_worker // _CHUNK_ROWS

    mesh = plsc.VectorSubcoreMesh(core_axis_name="c", subcore_axis_name="s")

    @functools.partial(
        pl.kernel,
        mesh=mesh,
        out_type=jax.ShapeDtypeStruct((batch, seq, emb), embeddings.dtype),
        scratch_types=[pltpu.VMEM((_CHUNK_ROWS, emb), embeddings.dtype)] * _N_BUF
        + [pltpu.SemaphoreType.DMA] * (2 * _N_BUF),
    )
    def sc_broadcast(table_hbm, out_hbm, *refs):
        bufs = refs[:_N_BUF]
        rsems = refs[_N_BUF : 2 * _N_BUF]
        wsems = refs[2 * _N_BUF :]
        wid = lax.axis_index("s") * info.num_cores + lax.axis_index("c")
        base = wid * rows_per_worker

        def chunk_slice(i):
            return pl.ds(base + i * _CHUNK_ROWS, _CHUNK_ROWS)

        pending_writes = [None] * n_chunks
        read_handles = [None] * n_chunks
        for j in range(min(_N_BUF, n_chunks)):
            read_handles[j] = pltpu.async_copy(
                table_hbm.at[chunk_slice(j)], bufs[j], rsems[j]
            )
        for i in range(n_chunks):
            k = i % _N_BUF
            read_handles[i].wait()
            pending_writes[i] = [
                pltpu.async_copy(bufs[k], out_hbm.at[b, chunk_slice(i)], wsems[k])
                for b in range(batch)
            ]
            m = i - 2
            if m >= 0 and m + _N_BUF < n_chunks:
                for h in pending_writes[m]:
                    h.wait()
                pending_writes[m] = None
                km = m % _N_BUF
                read_handles[m + _N_BUF] = pltpu.async_copy(
                    table_hbm.at[chunk_slice(m + _N_BUF)], bufs[km], rsems[km]
                )
        for i in range(n_chunks):
            if pending_writes[i] is not None:
                for h in pending_writes[i]:
                    h.wait()

    return sc_broadcast(embeddings)
